# baseline (speedup 1.0000x reference)
import functools

import jax
import jax.numpy as jnp
from jax import lax
from jax.experimental import pallas as pl
from jax.experimental.pallas import tpu as pltpu
from jax.experimental.pallas import tpu_sc as plsc

_DIM = 256
_NDIM = 3
_INV_SQRT2 = 0.7071067811865476

_G = 32768
_E = 262144
_NW = 32
_CPT = _G // _NW
_HC = _CPT // 2
_FP = 128
_CAP = 6144
_CH = 8192
_K = 128


def _seg_mean_body(gidx_hbm, m0_hbm, m1_hbm, out_hbm, idx_buf, lst,
                   gbuf, stage, accum, cnt, sem):
    c = lax.axis_index("c")
    s = lax.axis_index("s")
    wid = s * 2 + c
    base = wid * _CPT
    z16f = jnp.zeros((16,), jnp.float32)
    z16i = jnp.zeros((16,), jnp.int32)
    pad16 = jnp.full((16,), _HC << 18, jnp.int32)
    iota = lax.iota(jnp.int32, 16)
    onehot0 = jnp.where(iota == 0, 1, 0).astype(jnp.int32)
    _LS = _CAP + 32
    _CS = _HC + 32

    @plsc.parallel_loop(0, (2 * _LS) // 16, unroll=2)
    def pre(i):
        lst[pl.ds(i * 16, 16)] = pad16

    @plsc.parallel_loop(0, (2 * _CS) // 16)
    def zc(i):
        cnt[pl.ds(i * 16, 16)] = z16i

    def chunk_body(ci, offs):
        pltpu.sync_copy(gidx_hbm.at[pl.ds(ci * _CH, _CH)], idx_buf)

        @plsc.parallel_loop(0, _CH // 16, carry=offs)
        def vec_body(v, offs):
            off0, off1 = offs
            vec = idx_buf[pl.ds(v * 16, 16)]
            q = vec - base
            eid = ci * _CH + v * 16 + iota
            m0 = (q >= 0) & (q < _HC)
            pc0 = plsc.all_reduce_population_count(m0)
            inc0 = plsc.cumsum(m0.astype(jnp.int32))
            tgt0 = jnp.where(m0, off0 + inc0 - 1, _CAP + 16)
            plsc.store_scatter(lst, [tgt0], eid | (q << 18))
            off0 = jnp.minimum(off0 + pc0[0], _CAP)
            q1 = q - _HC
            m1 = (q1 >= 0) & (q1 < _HC)
            pc1 = plsc.all_reduce_population_count(m1)
            inc1 = plsc.cumsum(m1.astype(jnp.int32))
            tgt1 = jnp.where(m1, _LS + off1 + inc1 - 1, _CAP + 16)
            plsc.store_scatter(lst, [tgt1], eid | (q1 << 18))
            off1 = jnp.minimum(off1 + pc1[0], _CAP)
            return (off0, off1)

        return vec_body

    n0, n1 = lax.fori_loop(0, _E // _CH, chunk_body,
                           (jnp.int32(0), jnp.int32(0)))

    for p in range(2):
        m_hbm = (m0_hbm, m1_hbm)[p]
        count = p == 0

        def half_body(hh, _):
            lbase = hh * _LS
            cbase = hh * _CS
            n = jnp.where(hh == 0, n0, n1)
            nchunks = (n + _K - 1) // _K

            @plsc.parallel_loop(0, _HC + 1, unroll=2)
            def zr(i):
                for f in range(_FP // 16):
                    accum[i, pl.ds(f * 16, 16)] = z16f

            def fire(ci):
                so = (ci % 2) * _K
                for v in range(_K // 16):
                    pk = lst[pl.ds(lbase + ci * _K + v * 16, 16)]
                    gbuf[pl.ds(so + v * 16, 16)] = pk & 0x3FFFF
                pltpu.make_async_copy(
                    m_hbm.at[gbuf.at[pl.ds(so, _K)]],
                    stage.at[pl.ds(so, _K)], sem).start()

            def wait(ci):
                so = (ci % 2) * _K
                pltpu.make_async_copy(
                    m_hbm.at[gbuf.at[pl.ds(so, _K)]],
                    stage.at[pl.ds(so, _K)], sem).wait()

            def accumulate(ci):
                so = (ci % 2) * _K

                @plsc.parallel_loop(0, _K // 16)
                def grp_body(j16):
                    pkv = lst[pl.ds(lbase + ci * _K + j16 * 16, 16)]
                    qv = pkv >> 18
                    for l in range(16):
                        ql = qv[l]
                        for f in range(_FP // 16):
                            v = stage[so + j16 * 16 + l, pl.ds(f * 16, 16)]
                            plsc.addupdate(accum.at[ql, pl.ds(f * 16, 16)], v)
                        if count:
                            plsc.addupdate(cnt.at[pl.ds(cbase + ql, 16)],
                                           onehot0)

            @pl.when(nchunks > 0)
            def _():
                fire(0)

            def chunk_step(ci, _):
                @pl.when(ci + 1 < nchunks)
                def _():
                    fire(ci + 1)

                wait(ci)
                accumulate(ci)
                return 0

            lax.fori_loop(0, nchunks, chunk_step, 0)

            @plsc.parallel_loop(0, _HC // 16)
            def fin(cc16):
                cntv = cnt[pl.ds(cbase + cc16 * 16, 16)]
                rfv = 1.0 / jnp.maximum(cntv.astype(jnp.float32), 1.0)
                for l in range(16):
                    rf = rfv[l]
                    cc = cc16 * 16 + l
                    for f in range(_FP // 16):
                        accum[cc, pl.ds(f * 16, 16)] = (
                            accum[cc, pl.ds(f * 16, 16)] * rf)

            pltpu.sync_copy(
                accum.at[pl.ds(0, _HC)],
                out_hbm.at[pl.ds(base + hh * _HC, _HC), pl.ds(p * _FP, _FP)])
            return 0

        lax.fori_loop(0, 2, half_body, 0)


def _seg_mean(gidx, m0, m1):
    mesh = plsc.VectorSubcoreMesh(core_axis_name="c", subcore_axis_name="s")
    return pl.kernel(
        _seg_mean_body,
        out_type=jax.ShapeDtypeStruct((_G, _DIM), jnp.float32),
        mesh=mesh,
        compiler_params=pltpu.CompilerParams(needs_layout_passes=False),
        scratch_types=[
            pltpu.VMEM((_CH,), jnp.int32),
            pltpu.VMEM((2 * (_CAP + 32),), jnp.int32),
            pltpu.VMEM((2 * _K,), jnp.int32),
            pltpu.VMEM((2 * _K, _FP), jnp.float32),
            pltpu.VMEM((_HC + 1, _FP), jnp.float32),
            pltpu.VMEM((2 * (_HC + 32),), jnp.int32),
            pltpu.SemaphoreType.DMA,
        ],
    )(gidx, m0, m1)


def _gelu(x):
    return 0.5 * x * (1.0 + lax.erf(x * _INV_SQRT2))


def _sincos(coords, dim=_DIM, ndim=_NDIM, max_wavelength=10000.0):
    ndim_padding = dim % ndim
    dim_per_ndim = (dim - ndim_padding) // ndim
    sincos_padding = dim_per_ndim % 2
    padding = ndim_padding + sincos_padding * ndim
    eff = (dim - padding) // ndim
    half = eff // 2
    omega = 1.0 / (max_wavelength ** (jnp.arange(half, dtype=jnp.float32) / half))
    out = coords[:, :, None].astype(jnp.float32) * omega[None, None, :]
    emb = jnp.concatenate([jnp.sin(out), jnp.cos(out)], axis=-1)
    emb = emb.reshape(coords.shape[0], ndim * eff)
    if padding > 0:
        emb = jnp.pad(emb, ((0, 0), (0, padding)))
    return emb


def _edge_mlp_body(xa_ref, xb_ref, w1a_ref, w1b_ref, b1_ref,
                   w2_ref, b2_ref, w3_ref, b3_ref, o_ref):
    h = (jnp.dot(xa_ref[...], w1a_ref[...], preferred_element_type=jnp.float32)
         + jnp.dot(xb_ref[...], w1b_ref[...], preferred_element_type=jnp.float32)
         + b1_ref[...])
    h = _gelu(h)
    h = _gelu(jnp.dot(h, w2_ref[...], preferred_element_type=jnp.float32)
              + b2_ref[...])
    o_ref[...] = (jnp.dot(h, w3_ref[...], preferred_element_type=jnp.float32)
                  + b3_ref[...])


def _edge_mlp(xa, xb, w1a, w1b, b1, w2, b2, w3, b3, block_e=2048):
    e = xa.shape[0]
    d = _DIM
    grid = (e // block_e,)
    full = lambda shape: pl.BlockSpec(shape, lambda i: (0, 0))
    return pl.pallas_call(
        _edge_mlp_body,
        grid=grid,
        in_specs=[
            pl.BlockSpec((block_e, d), lambda i: (i, 0)),
            pl.BlockSpec((block_e, d), lambda i: (i, 0)),
            full((d, 2 * d)),
            full((d, 2 * d)),
            full((1, 2 * d)),
            full((2 * d, d)),
            full((1, d)),
            full((d, d)),
            full((1, d)),
        ],
        out_specs=pl.BlockSpec((block_e, d), lambda i: (i, 0)),
        out_shape=jax.ShapeDtypeStruct((e, d), jnp.float32),
    )(xa, xb, w1a, w1b, b1.reshape(1, -1), w2, b2.reshape(1, -1),
      w3, b3.reshape(1, -1))


def kernel(mesh_pos, sdf, grid_pos, mesh_to_grid_edges,
           sdf_w1, sdf_b1, sdf_w2, sdf_b2,
           msg_w1, msg_b1, msg_w2, msg_b2, msg_w3, msg_b3):
    g = grid_pos.shape[0]
    mesh_e = _sincos(mesh_pos)
    grid_pe = _sincos(grid_pos)
    s = sdf.reshape(-1, 1)
    s = _gelu(s @ sdf_w1 + sdf_b1) @ sdf_w2 + sdf_b2
    grid_embed = grid_pe + s

    w1a = msg_w1[:_DIM]
    w1b = msg_w1[_DIM:]
    a, b = _precompute(mesh_e, grid_embed, w1a, w1b, msg_b1)

    grid_idx = mesh_to_grid_edges[:, 0]
    mesh_idx = mesh_to_grid_edges[:, 1]
    xa = jnp.take(a, mesh_idx, axis=0)
    xb = jnp.take(b, grid_idx, axis=0)

    m0, m1 = _edge_mlp_pre(xa, xb, msg_w2, msg_b2, msg_w3, msg_b3)

    mean = _seg_mean(grid_idx, m0, m1)
    return mean.reshape(1, g, _DIM)


def _edge_mlp_pre_body(xa_ref, xb_ref, w2_ref, b2_ref,
                       w3_ref, b3_ref, o0_ref, o1_ref):
    h = _gelu(xa_ref[...].astype(jnp.float32) + xb_ref[...].astype(jnp.float32))
    h = _gelu(jnp.dot(h.astype(jnp.bfloat16), w2_ref[...],
                      preferred_element_type=jnp.float32) + b2_ref[...])
    o = (jnp.dot(h.astype(jnp.bfloat16), w3_ref[...],
                 preferred_element_type=jnp.float32) + b3_ref[...])
    o0_ref[...] = o[:, :_FP]
    o1_ref[...] = o[:, _FP:]


def _edge_mlp_pre(xa, xb, w2, b2, w3, b3, block_e=2048):
    e = xa.shape[0]
    d = _DIM
    full = lambda shape: pl.BlockSpec(shape, lambda i: (0, 0))
    return pl.pallas_call(
        _edge_mlp_pre_body,
        grid=(e // block_e,),
        in_specs=[
            pl.BlockSpec((block_e, 2 * d), lambda i: (i, 0)),
            pl.BlockSpec((block_e, 2 * d), lambda i: (i, 0)),
            full((2 * d, d)),
            full((1, d)),
            full((d, d)),
            full((1, d)),
        ],
        out_specs=[pl.BlockSpec((block_e, _FP), lambda i: (i, 0)),
                   pl.BlockSpec((block_e, _FP), lambda i: (i, 0))],
        out_shape=[jax.ShapeDtypeStruct((e, _FP), jnp.float32),
                   jax.ShapeDtypeStruct((e, _FP), jnp.float32)],
    )(xa, xb, w2.astype(jnp.bfloat16), b2.reshape(1, -1),
      w3.astype(jnp.bfloat16), b3.reshape(1, -1))


def _precompute_body(me_ref, ge_ref, w1a_ref, w1b_ref, b1_ref, a_ref, b_ref):
    a_ref[...] = jnp.dot(me_ref[...].astype(jnp.bfloat16), w1a_ref[...],
                         preferred_element_type=jnp.float32
                         ).astype(jnp.bfloat16)
    b_ref[...] = (jnp.dot(ge_ref[...].astype(jnp.bfloat16), w1b_ref[...],
                          preferred_element_type=jnp.float32)
                  + b1_ref[...]).astype(jnp.bfloat16)


def _precompute(mesh_e, grid_embed, w1a, w1b, b1, block_n=2048):
    n = mesh_e.shape[0]
    d = _DIM
    full = lambda shape: pl.BlockSpec(shape, lambda i: (0, 0))
    return pl.pallas_call(
        _precompute_body,
        grid=(n // block_n,),
        in_specs=[
            pl.BlockSpec((block_n, d), lambda i: (i, 0)),
            pl.BlockSpec((block_n, d), lambda i: (i, 0)),
            full((d, 2 * d)),
            full((d, 2 * d)),
            full((1, 2 * d)),
        ],
        out_specs=[pl.BlockSpec((block_n, 2 * d), lambda i: (i, 0)),
                   pl.BlockSpec((block_n, 2 * d), lambda i: (i, 0))],
        out_shape=[jax.ShapeDtypeStruct((n, 2 * d), jnp.bfloat16),
                   jax.ShapeDtypeStruct((n, 2 * d), jnp.bfloat16)],
    )(mesh_e, grid_embed, w1a.astype(jnp.bfloat16), w1b.astype(jnp.bfloat16),
      b1.reshape(1, -1))


# speedup vs baseline: 1.5086x; 1.5086x over previous
import functools

import jax
import jax.numpy as jnp
from jax import lax
from jax.experimental import pallas as pl
from jax.experimental.pallas import tpu as pltpu
from jax.experimental.pallas import tpu_sc as plsc

_DIM = 256
_NDIM = 3
_INV_SQRT2 = 0.7071067811865476

_G = 32768
_E = 262144
_NW = 32
_CPT = _G // _NW
_HC = _CPT // 2
_FP = 128
_CAP = 6144
_CH = 8192
_K = 128


def _seg_mean_body(gidx_hbm, m0_hbm, m1_hbm, out_hbm, idx_buf, lst,
                   gbuf, stage, accum, cnt, sem):
    c = lax.axis_index("c")
    s = lax.axis_index("s")
    wid = s * 2 + c
    base = wid * _CPT
    z16f = jnp.zeros((16,), jnp.float32)
    z16i = jnp.zeros((16,), jnp.int32)
    pad16 = jnp.full((16,), _HC << 18, jnp.int32)
    iota = lax.iota(jnp.int32, 16)
    onehot0 = jnp.where(iota == 0, 1, 0).astype(jnp.int32)
    _LS = _CAP + 32
    _CS = _HC + 32

    @plsc.parallel_loop(0, (2 * _LS) // 16, unroll=2)
    def pre(i):
        lst[pl.ds(i * 16, 16)] = pad16

    @plsc.parallel_loop(0, (2 * _CS) // 16)
    def zc(i):
        cnt[pl.ds(i * 16, 16)] = z16i

    def chunk_body(ci, offs):
        pltpu.sync_copy(gidx_hbm.at[pl.ds(ci * _CH, _CH)], idx_buf)

        @plsc.parallel_loop(0, _CH // 16, carry=offs)
        def vec_body(v, offs):
            off0, off1 = offs
            vec = idx_buf[pl.ds(v * 16, 16)]
            q = vec - base
            eid = ci * _CH + v * 16 + iota
            m0 = (q >= 0) & (q < _HC)
            pc0 = plsc.all_reduce_population_count(m0)
            inc0 = plsc.cumsum(m0.astype(jnp.int32))
            tgt0 = jnp.where(m0, off0 + inc0 - 1, _CAP + 16)
            plsc.store_scatter(lst, [tgt0], eid | (q << 18))
            off0 = jnp.minimum(off0 + pc0[0], _CAP)
            q1 = q - _HC
            m1 = (q1 >= 0) & (q1 < _HC)
            pc1 = plsc.all_reduce_population_count(m1)
            inc1 = plsc.cumsum(m1.astype(jnp.int32))
            tgt1 = jnp.where(m1, _LS + off1 + inc1 - 1, _CAP + 16)
            plsc.store_scatter(lst, [tgt1], eid | (q1 << 18))
            off1 = jnp.minimum(off1 + pc1[0], _CAP)
            return (off0, off1)

        return vec_body

    n0, n1 = lax.fori_loop(0, _E // _CH, chunk_body,
                           (jnp.int32(0), jnp.int32(0)))

    for p in range(2):
        m_hbm = (m0_hbm, m1_hbm)[p]
        count = p == 0

        def half_body(hh, _):
            lbase = hh * _LS
            cbase = hh * _CS
            n = jnp.where(hh == 0, n0, n1)
            nchunks = (n + _K - 1) // _K

            @plsc.parallel_loop(0, _HC + 1, unroll=2)
            def zr(i):
                for f in range(_FP // 16):
                    accum[i, pl.ds(f * 16, 16)] = z16f

            def fire(ci):
                so = (ci % 2) * _K
                for v in range(_K // 16):
                    pk = lst[pl.ds(lbase + ci * _K + v * 16, 16)]
                    gbuf[pl.ds(so + v * 16, 16)] = pk & 0x3FFFF
                pltpu.make_async_copy(
                    m_hbm.at[gbuf.at[pl.ds(so, _K)]],
                    stage.at[pl.ds(so, _K)], sem).start()

            def wait(ci):
                so = (ci % 2) * _K
                pltpu.make_async_copy(
                    m_hbm.at[gbuf.at[pl.ds(so, _K)]],
                    stage.at[pl.ds(so, _K)], sem).wait()

            def accumulate(ci):
                so = (ci % 2) * _K

                @plsc.parallel_loop(0, _K // 16)
                def grp_body(j16):
                    pkv = lst[pl.ds(lbase + ci * _K + j16 * 16, 16)]
                    qv = pkv >> 18
                    for l in range(16):
                        ql = qv[l]
                        for f in range(_FP // 16):
                            v = stage[so + j16 * 16 + l, pl.ds(f * 16, 16)]
                            plsc.addupdate(accum.at[ql, pl.ds(f * 16, 16)], v)
                        if count:
                            plsc.addupdate(cnt.at[pl.ds(cbase + ql, 16)],
                                           onehot0)

            @pl.when(nchunks > 0)
            def _():
                fire(0)

            def chunk_step(ci, _):
                @pl.when(ci + 1 < nchunks)
                def _():
                    fire(ci + 1)

                wait(ci)
                accumulate(ci)
                return 0

            lax.fori_loop(0, nchunks, chunk_step, 0)

            @plsc.parallel_loop(0, _HC // 16)
            def fin(cc16):
                cntv = cnt[pl.ds(cbase + cc16 * 16, 16)]
                rfv = 1.0 / jnp.maximum(cntv.astype(jnp.float32), 1.0)
                for l in range(16):
                    rf = rfv[l]
                    cc = cc16 * 16 + l
                    for f in range(_FP // 16):
                        accum[cc, pl.ds(f * 16, 16)] = (
                            accum[cc, pl.ds(f * 16, 16)] * rf)

            pltpu.sync_copy(
                accum.at[pl.ds(0, _HC)],
                out_hbm.at[pl.ds(base + hh * _HC, _HC), pl.ds(p * _FP, _FP)])
            return 0

        lax.fori_loop(0, 2, half_body, 0)


def _seg_mean(gidx, m0, m1):
    mesh = plsc.VectorSubcoreMesh(core_axis_name="c", subcore_axis_name="s")
    return pl.kernel(
        _seg_mean_body,
        out_type=jax.ShapeDtypeStruct((_G, _DIM), jnp.float32),
        mesh=mesh,
        compiler_params=pltpu.CompilerParams(needs_layout_passes=False),
        scratch_types=[
            pltpu.VMEM((_CH,), jnp.int32),
            pltpu.VMEM((2 * (_CAP + 32),), jnp.int32),
            pltpu.VMEM((2 * _K,), jnp.int32),
            pltpu.VMEM((2 * _K, _FP), jnp.float32),
            pltpu.VMEM((_HC + 1, _FP), jnp.float32),
            pltpu.VMEM((2 * (_HC + 32),), jnp.int32),
            pltpu.SemaphoreType.DMA,
        ],
    )(gidx, m0, m1)


def _gelu(x):
    return 0.5 * x * (1.0 + lax.erf(x * _INV_SQRT2))


def _sincos(coords, dim=_DIM, ndim=_NDIM, max_wavelength=10000.0):
    ndim_padding = dim % ndim
    dim_per_ndim = (dim - ndim_padding) // ndim
    sincos_padding = dim_per_ndim % 2
    padding = ndim_padding + sincos_padding * ndim
    eff = (dim - padding) // ndim
    half = eff // 2
    omega = 1.0 / (max_wavelength ** (jnp.arange(half, dtype=jnp.float32) / half))
    out = coords[:, :, None].astype(jnp.float32) * omega[None, None, :]
    emb = jnp.concatenate([jnp.sin(out), jnp.cos(out)], axis=-1)
    emb = emb.reshape(coords.shape[0], ndim * eff)
    if padding > 0:
        emb = jnp.pad(emb, ((0, 0), (0, padding)))
    return emb


def _edge_mlp_body(xa_ref, xb_ref, w1a_ref, w1b_ref, b1_ref,
                   w2_ref, b2_ref, w3_ref, b3_ref, o_ref):
    h = (jnp.dot(xa_ref[...], w1a_ref[...], preferred_element_type=jnp.float32)
         + jnp.dot(xb_ref[...], w1b_ref[...], preferred_element_type=jnp.float32)
         + b1_ref[...])
    h = _gelu(h)
    h = _gelu(jnp.dot(h, w2_ref[...], preferred_element_type=jnp.float32)
              + b2_ref[...])
    o_ref[...] = (jnp.dot(h, w3_ref[...], preferred_element_type=jnp.float32)
                  + b3_ref[...])


def _edge_mlp(xa, xb, w1a, w1b, b1, w2, b2, w3, b3, block_e=2048):
    e = xa.shape[0]
    d = _DIM
    grid = (e // block_e,)
    full = lambda shape: pl.BlockSpec(shape, lambda i: (0, 0))
    return pl.pallas_call(
        _edge_mlp_body,
        grid=grid,
        in_specs=[
            pl.BlockSpec((block_e, d), lambda i: (i, 0)),
            pl.BlockSpec((block_e, d), lambda i: (i, 0)),
            full((d, 2 * d)),
            full((d, 2 * d)),
            full((1, 2 * d)),
            full((2 * d, d)),
            full((1, d)),
            full((d, d)),
            full((1, d)),
        ],
        out_specs=pl.BlockSpec((block_e, d), lambda i: (i, 0)),
        out_shape=jax.ShapeDtypeStruct((e, d), jnp.float32),
    )(xa, xb, w1a, w1b, b1.reshape(1, -1), w2, b2.reshape(1, -1),
      w3, b3.reshape(1, -1))


def kernel(mesh_pos, sdf, grid_pos, mesh_to_grid_edges,
           sdf_w1, sdf_b1, sdf_w2, sdf_b2,
           msg_w1, msg_b1, msg_w2, msg_b2, msg_w3, msg_b3):
    g = grid_pos.shape[0]
    mesh_e = _sincos(mesh_pos)
    grid_pe = _sincos(grid_pos)
    s = sdf.reshape(-1, 1)
    s = _gelu(s @ sdf_w1 + sdf_b1) @ sdf_w2 + sdf_b2
    grid_embed = grid_pe + s

    w1a = msg_w1[:_DIM]
    w1b = msg_w1[_DIM:]
    a, b = _precompute(mesh_e, grid_embed, w1a, w1b, msg_b1)

    grid_idx = mesh_to_grid_edges[:, 0]
    mesh_idx = mesh_to_grid_edges[:, 1]

    def _take32(t, idx):
        n = t.shape[0]
        t32 = lax.bitcast_convert_type(t.reshape(n, _DIM, 2), jnp.int32)
        g32 = jnp.take(t32, idx, axis=0)
        return lax.bitcast_convert_type(g32, jnp.bfloat16).reshape(-1, 2 * _DIM)

    xa = _take32(a, mesh_idx)
    xb = _take32(b, grid_idx)

    m0, m1 = _edge_mlp_pre(xa, xb, msg_w2, msg_b2, msg_w3, msg_b3)

    mean = _seg_mean(grid_idx, m0, m1)
    return mean.reshape(1, g, _DIM)


def _edge_mlp_pre_body(xa_ref, xb_ref, w2_ref, b2_ref,
                       w3_ref, b3_ref, o0_ref, o1_ref):
    h = _gelu(xa_ref[...].astype(jnp.float32) + xb_ref[...].astype(jnp.float32))
    h = _gelu(jnp.dot(h.astype(jnp.bfloat16), w2_ref[...],
                      preferred_element_type=jnp.float32) + b2_ref[...])
    o = (jnp.dot(h.astype(jnp.bfloat16), w3_ref[...],
                 preferred_element_type=jnp.float32) + b3_ref[...])
    o0_ref[...] = o[:, :_FP]
    o1_ref[...] = o[:, _FP:]


def _edge_mlp_pre(xa, xb, w2, b2, w3, b3, block_e=2048):
    e = xa.shape[0]
    d = _DIM
    full = lambda shape: pl.BlockSpec(shape, lambda i: (0, 0))
    return pl.pallas_call(
        _edge_mlp_pre_body,
        grid=(e // block_e,),
        in_specs=[
            pl.BlockSpec((block_e, 2 * d), lambda i: (i, 0)),
            pl.BlockSpec((block_e, 2 * d), lambda i: (i, 0)),
            full((2 * d, d)),
            full((1, d)),
            full((d, d)),
            full((1, d)),
        ],
        out_specs=[pl.BlockSpec((block_e, _FP), lambda i: (i, 0)),
                   pl.BlockSpec((block_e, _FP), lambda i: (i, 0))],
        out_shape=[jax.ShapeDtypeStruct((e, _FP), jnp.float32),
                   jax.ShapeDtypeStruct((e, _FP), jnp.float32)],
    )(xa, xb, w2.astype(jnp.bfloat16), b2.reshape(1, -1),
      w3.astype(jnp.bfloat16), b3.reshape(1, -1))


def _precompute_body(me_ref, ge_ref, w1a_ref, w1b_ref, b1_ref, a_ref, b_ref):
    a_ref[...] = jnp.dot(me_ref[...].astype(jnp.bfloat16), w1a_ref[...],
                         preferred_element_type=jnp.float32
                         ).astype(jnp.bfloat16)
    b_ref[...] = (jnp.dot(ge_ref[...].astype(jnp.bfloat16), w1b_ref[...],
                          preferred_element_type=jnp.float32)
                  + b1_ref[...]).astype(jnp.bfloat16)


def _precompute(mesh_e, grid_embed, w1a, w1b, b1, block_n=2048):
    n = mesh_e.shape[0]
    d = _DIM
    full = lambda shape: pl.BlockSpec(shape, lambda i: (0, 0))
    return pl.pallas_call(
        _precompute_body,
        grid=(n // block_n,),
        in_specs=[
            pl.BlockSpec((block_n, d), lambda i: (i, 0)),
            pl.BlockSpec((block_n, d), lambda i: (i, 0)),
            full((d, 2 * d)),
            full((d, 2 * d)),
            full((1, 2 * d)),
        ],
        out_specs=[pl.BlockSpec((block_n, 2 * d), lambda i: (i, 0)),
                   pl.BlockSpec((block_n, 2 * d), lambda i: (i, 0))],
        out_shape=[jax.ShapeDtypeStruct((n, 2 * d), jnp.bfloat16),
                   jax.ShapeDtypeStruct((n, 2 * d), jnp.bfloat16)],
    )(mesh_e, grid_embed, w1a.astype(jnp.bfloat16), w1b.astype(jnp.bfloat16),
      b1.reshape(1, -1))


# speedup vs baseline: 3.2219x; 2.1357x over previous
import functools

import jax
import jax.numpy as jnp
from jax import lax
from jax.experimental import pallas as pl
from jax.experimental.pallas import tpu as pltpu
from jax.experimental.pallas import tpu_sc as plsc

_DIM = 256
_NDIM = 3
_INV_SQRT2 = 0.7071067811865476

_G = 32768
_E = 262144
_NW = 32
_CPT = _G // _NW
_HC = _CPT // 2
_FP = 128
_CAP = 6144
_CH = 8192
_K = 128


def _seg_mean_body(gidx_hbm, m0_hbm, m1_hbm, out_hbm, idx_buf, lst,
                   gbuf, stage, accum, cnt, sem):
    c = lax.axis_index("c")
    s = lax.axis_index("s")
    wid = s * 2 + c
    base = wid * _CPT
    z16f = jnp.zeros((16,), jnp.float32)
    z16i = jnp.zeros((16,), jnp.int32)
    pad16 = jnp.full((16,), _HC << 18, jnp.int32)
    iota = lax.iota(jnp.int32, 16)
    onehot0 = jnp.where(iota == 0, 1, 0).astype(jnp.int32)
    _LS = _CAP + 32
    _CS = _HC + 32

    @plsc.parallel_loop(0, (2 * _LS) // 16, unroll=2)
    def pre(i):
        lst[pl.ds(i * 16, 16)] = pad16

    @plsc.parallel_loop(0, (2 * _CS) // 16)
    def zc(i):
        cnt[pl.ds(i * 16, 16)] = z16i

    def chunk_body(ci, offs):
        pltpu.sync_copy(gidx_hbm.at[pl.ds(ci * _CH, _CH)], idx_buf)

        @plsc.parallel_loop(0, _CH // 16, carry=offs)
        def vec_body(v, offs):
            off0, off1 = offs
            vec = idx_buf[pl.ds(v * 16, 16)]
            q = vec - base
            eid = ci * _CH + v * 16 + iota
            m0 = (q >= 0) & (q < _HC)
            pc0 = plsc.all_reduce_population_count(m0)
            inc0 = plsc.cumsum(m0.astype(jnp.int32))
            tgt0 = jnp.where(m0, off0 + inc0 - 1, _CAP + 16)
            plsc.store_scatter(lst, [tgt0], eid | (q << 18))
            off0 = jnp.minimum(off0 + pc0[0], _CAP)
            q1 = q - _HC
            m1 = (q1 >= 0) & (q1 < _HC)
            pc1 = plsc.all_reduce_population_count(m1)
            inc1 = plsc.cumsum(m1.astype(jnp.int32))
            tgt1 = jnp.where(m1, _LS + off1 + inc1 - 1, _CAP + 16)
            plsc.store_scatter(lst, [tgt1], eid | (q1 << 18))
            off1 = jnp.minimum(off1 + pc1[0], _CAP)
            return (off0, off1)

        return vec_body

    n0, n1 = lax.fori_loop(0, _E // _CH, chunk_body,
                           (jnp.int32(0), jnp.int32(0)))

    for p in range(2):
        m_hbm = (m0_hbm, m1_hbm)[p]
        count = p == 0

        def half_body(hh, _):
            lbase = hh * _LS
            cbase = hh * _CS
            n = jnp.where(hh == 0, n0, n1)
            nchunks = (n + _K - 1) // _K

            @plsc.parallel_loop(0, _HC + 1, unroll=2)
            def zr(i):
                for f in range(_FP // 16):
                    accum[i, pl.ds(f * 16, 16)] = z16f

            def fire(ci):
                so = (ci % 2) * _K
                for v in range(_K // 16):
                    pk = lst[pl.ds(lbase + ci * _K + v * 16, 16)]
                    gbuf[pl.ds(so + v * 16, 16)] = pk & 0x3FFFF
                pltpu.make_async_copy(
                    m_hbm.at[gbuf.at[pl.ds(so, _K)]],
                    stage.at[pl.ds(so, _K)], sem).start()

            def wait(ci):
                so = (ci % 2) * _K
                pltpu.make_async_copy(
                    m_hbm.at[gbuf.at[pl.ds(so, _K)]],
                    stage.at[pl.ds(so, _K)], sem).wait()

            def accumulate(ci):
                so = (ci % 2) * _K

                @plsc.parallel_loop(0, _K // 16)
                def grp_body(j16):
                    pkv = lst[pl.ds(lbase + ci * _K + j16 * 16, 16)]
                    qv = pkv >> 18
                    for l in range(16):
                        ql = qv[l]
                        for f in range(_FP // 16):
                            v = stage[so + j16 * 16 + l, pl.ds(f * 16, 16)]
                            plsc.addupdate(accum.at[ql, pl.ds(f * 16, 16)], v)
                        if count:
                            plsc.addupdate(cnt.at[pl.ds(cbase + ql, 16)],
                                           onehot0)

            @pl.when(nchunks > 0)
            def _():
                fire(0)

            def chunk_step(ci, _):
                @pl.when(ci + 1 < nchunks)
                def _():
                    fire(ci + 1)

                wait(ci)
                accumulate(ci)
                return 0

            lax.fori_loop(0, nchunks, chunk_step, 0)

            @plsc.parallel_loop(0, _HC // 16)
            def fin(cc16):
                cntv = cnt[pl.ds(cbase + cc16 * 16, 16)]
                rfv = 1.0 / jnp.maximum(cntv.astype(jnp.float32), 1.0)
                for l in range(16):
                    rf = rfv[l]
                    cc = cc16 * 16 + l
                    for f in range(_FP // 16):
                        accum[cc, pl.ds(f * 16, 16)] = (
                            accum[cc, pl.ds(f * 16, 16)] * rf)

            pltpu.sync_copy(
                accum.at[pl.ds(0, _HC)],
                out_hbm.at[pl.ds(base + hh * _HC, _HC), pl.ds(p * _FP, _FP)])
            return 0

        lax.fori_loop(0, 2, half_body, 0)


def _seg_mean(gidx, m0, m1):
    mesh = plsc.VectorSubcoreMesh(core_axis_name="c", subcore_axis_name="s")
    return pl.kernel(
        _seg_mean_body,
        out_type=jax.ShapeDtypeStruct((_G, _DIM), jnp.float32),
        mesh=mesh,
        compiler_params=pltpu.CompilerParams(needs_layout_passes=False),
        scratch_types=[
            pltpu.VMEM((_CH,), jnp.int32),
            pltpu.VMEM((2 * (_CAP + 32),), jnp.int32),
            pltpu.VMEM((2 * _K,), jnp.int32),
            pltpu.VMEM((2 * _K, _FP), jnp.float32),
            pltpu.VMEM((_HC + 1, _FP), jnp.float32),
            pltpu.VMEM((2 * (_HC + 32),), jnp.int32),
            pltpu.SemaphoreType.DMA,
        ],
    )(gidx, m0, m1)


def _gelu(x):
    return 0.5 * x * (1.0 + lax.erf(x * _INV_SQRT2))


def _sincos(coords, dim=_DIM, ndim=_NDIM, max_wavelength=10000.0):
    ndim_padding = dim % ndim
    dim_per_ndim = (dim - ndim_padding) // ndim
    sincos_padding = dim_per_ndim % 2
    padding = ndim_padding + sincos_padding * ndim
    eff = (dim - padding) // ndim
    half = eff // 2
    omega = 1.0 / (max_wavelength ** (jnp.arange(half, dtype=jnp.float32) / half))
    out = coords[:, :, None].astype(jnp.float32) * omega[None, None, :]
    emb = jnp.concatenate([jnp.sin(out), jnp.cos(out)], axis=-1)
    emb = emb.reshape(coords.shape[0], ndim * eff)
    if padding > 0:
        emb = jnp.pad(emb, ((0, 0), (0, padding)))
    return emb


def _edge_mlp_body(xa_ref, xb_ref, w1a_ref, w1b_ref, b1_ref,
                   w2_ref, b2_ref, w3_ref, b3_ref, o_ref):
    h = (jnp.dot(xa_ref[...], w1a_ref[...], preferred_element_type=jnp.float32)
         + jnp.dot(xb_ref[...], w1b_ref[...], preferred_element_type=jnp.float32)
         + b1_ref[...])
    h = _gelu(h)
    h = _gelu(jnp.dot(h, w2_ref[...], preferred_element_type=jnp.float32)
              + b2_ref[...])
    o_ref[...] = (jnp.dot(h, w3_ref[...], preferred_element_type=jnp.float32)
                  + b3_ref[...])


def _edge_mlp(xa, xb, w1a, w1b, b1, w2, b2, w3, b3, block_e=2048):
    e = xa.shape[0]
    d = _DIM
    grid = (e // block_e,)
    full = lambda shape: pl.BlockSpec(shape, lambda i: (0, 0))
    return pl.pallas_call(
        _edge_mlp_body,
        grid=grid,
        in_specs=[
            pl.BlockSpec((block_e, d), lambda i: (i, 0)),
            pl.BlockSpec((block_e, d), lambda i: (i, 0)),
            full((d, 2 * d)),
            full((d, 2 * d)),
            full((1, 2 * d)),
            full((2 * d, d)),
            full((1, d)),
            full((d, d)),
            full((1, d)),
        ],
        out_specs=pl.BlockSpec((block_e, d), lambda i: (i, 0)),
        out_shape=jax.ShapeDtypeStruct((e, d), jnp.float32),
    )(xa, xb, w1a, w1b, b1.reshape(1, -1), w2, b2.reshape(1, -1),
      w3, b3.reshape(1, -1))


def kernel(mesh_pos, sdf, grid_pos, mesh_to_grid_edges,
           sdf_w1, sdf_b1, sdf_w2, sdf_b2,
           msg_w1, msg_b1, msg_w2, msg_b2, msg_w3, msg_b3):
    g = grid_pos.shape[0]
    mesh_e = _sincos(mesh_pos)
    grid_pe = _sincos(grid_pos)
    s = sdf.reshape(-1, 1)
    s = _gelu(s @ sdf_w1 + sdf_b1) @ sdf_w2 + sdf_b2
    grid_embed = grid_pe + s

    w1a = msg_w1[:_DIM]
    w1b = msg_w1[_DIM:]
    a, b = _precompute(mesh_e, grid_embed, w1a, w1b, msg_b1)

    grid_idx = mesh_to_grid_edges[:, 0]
    mesh_idx = mesh_to_grid_edges[:, 1]

    xa = jnp.take(a, mesh_idx, axis=0)
    xb = jnp.take(b, grid_idx, axis=0)

    m0, m1 = _edge_mlp_pre(xa, xb, msg_w2, msg_b2, msg_w3, msg_b3)

    mean = _seg_mean(grid_idx, m0, m1)
    return mean.reshape(1, g, _DIM)


def _edge_mlp_pre_body(xa_ref, xb_ref, w2_ref, b2_ref,
                       w3_ref, b3_ref, o0_ref, o1_ref):
    alo, ahi = _unpack32(xa_ref[...])
    blo, bhi = _unpack32(xb_ref[...])
    h = _gelu(jnp.concatenate([alo + blo, ahi + bhi], axis=1))
    h = _gelu(jnp.dot(h.astype(jnp.bfloat16), w2_ref[...],
                      preferred_element_type=jnp.float32) + b2_ref[...])
    o = (jnp.dot(h.astype(jnp.bfloat16), w3_ref[...],
                 preferred_element_type=jnp.float32) + b3_ref[...])
    o0_ref[...] = o[:, :_FP]
    o1_ref[...] = o[:, _FP:]


def _edge_mlp_pre(xa, xb, w2, b2, w3, b3, block_e=2048):
    e = xa.shape[0]
    d = _DIM
    full = lambda shape: pl.BlockSpec(shape, lambda i: (0, 0))
    return pl.pallas_call(
        _edge_mlp_pre_body,
        grid=(e // block_e,),
        in_specs=[
            pl.BlockSpec((block_e, d), lambda i: (i, 0)),
            pl.BlockSpec((block_e, d), lambda i: (i, 0)),
            full((2 * d, d)),
            full((1, d)),
            full((d, d)),
            full((1, d)),
        ],
        out_specs=[pl.BlockSpec((block_e, _FP), lambda i: (i, 0)),
                   pl.BlockSpec((block_e, _FP), lambda i: (i, 0))],
        out_shape=[jax.ShapeDtypeStruct((e, _FP), jnp.float32),
                   jax.ShapeDtypeStruct((e, _FP), jnp.float32)],
    )(xa, xb, w2.astype(jnp.bfloat16), b2.reshape(1, -1),
      w3.astype(jnp.bfloat16), b3.reshape(1, -1))


def _pack32(x):
    lo = lax.bitcast_convert_type(x[:, :_DIM], jnp.int16).astype(jnp.int32)
    hi = lax.bitcast_convert_type(x[:, _DIM:], jnp.int16).astype(jnp.int32)
    return (lo & 0xFFFF) | (hi << 16)


def _unpack32(w):
    lo = lax.bitcast_convert_type(w.astype(jnp.int16), jnp.bfloat16)
    hi = lax.bitcast_convert_type((w >> 16).astype(jnp.int16), jnp.bfloat16)
    return lo.astype(jnp.float32), hi.astype(jnp.float32)


def _precompute_body(me_ref, ge_ref, w1a_ref, w1b_ref, b1_ref, a_ref, b_ref):
    a = jnp.dot(me_ref[...].astype(jnp.bfloat16), w1a_ref[...],
                preferred_element_type=jnp.float32).astype(jnp.bfloat16)
    a_ref[...] = _pack32(a)
    b = (jnp.dot(ge_ref[...].astype(jnp.bfloat16), w1b_ref[...],
                 preferred_element_type=jnp.float32)
         + b1_ref[...]).astype(jnp.bfloat16)
    b_ref[...] = _pack32(b)


def _precompute(mesh_e, grid_embed, w1a, w1b, b1, block_n=2048):
    n = mesh_e.shape[0]
    d = _DIM
    full = lambda shape: pl.BlockSpec(shape, lambda i: (0, 0))
    return pl.pallas_call(
        _precompute_body,
        grid=(n // block_n,),
        in_specs=[
            pl.BlockSpec((block_n, d), lambda i: (i, 0)),
            pl.BlockSpec((block_n, d), lambda i: (i, 0)),
            full((d, 2 * d)),
            full((d, 2 * d)),
            full((1, 2 * d)),
        ],
        out_specs=[pl.BlockSpec((block_n, d), lambda i: (i, 0)),
                   pl.BlockSpec((block_n, d), lambda i: (i, 0))],
        out_shape=[jax.ShapeDtypeStruct((n, d), jnp.int32),
                   jax.ShapeDtypeStruct((n, d), jnp.int32)],
    )(mesh_e, grid_embed, w1a.astype(jnp.bfloat16), w1b.astype(jnp.bfloat16),
      b1.reshape(1, -1))


# speedup vs baseline: 3.2248x; 1.0009x over previous
import functools

import jax
import jax.numpy as jnp
from jax import lax
from jax.experimental import pallas as pl
from jax.experimental.pallas import tpu as pltpu
from jax.experimental.pallas import tpu_sc as plsc

_DIM = 256
_NDIM = 3
_INV_SQRT2 = 0.7071067811865476

_G = 32768
_E = 262144
_NW = 32
_CPT = _G // _NW
_HC = _CPT // 2
_FP = 128
_CAP = 6144
_CH = 8192
_K = 128


def _seg_mean_body(gidx_hbm, m0_hbm, m1_hbm, out_hbm, idx_buf, lst,
                   gbuf, stage, accum, cnt, sem):
    c = lax.axis_index("c")
    s = lax.axis_index("s")
    wid = s * 2 + c
    base = wid * _CPT
    z16f = jnp.zeros((16,), jnp.float32)
    z16i = jnp.zeros((16,), jnp.int32)
    pad16 = jnp.full((16,), _HC << 18, jnp.int32)
    iota = lax.iota(jnp.int32, 16)
    onehot0 = jnp.where(iota == 0, 1, 0).astype(jnp.int32)
    _LS = _CAP + 32
    _CS = _HC + 32

    @plsc.parallel_loop(0, (2 * _LS) // 16, unroll=2)
    def pre(i):
        lst[pl.ds(i * 16, 16)] = pad16

    @plsc.parallel_loop(0, (2 * _CS) // 16)
    def zc(i):
        cnt[pl.ds(i * 16, 16)] = z16i

    def chunk_body(ci, offs):
        pltpu.sync_copy(gidx_hbm.at[pl.ds(ci * _CH, _CH)], idx_buf)

        @plsc.parallel_loop(0, _CH // 16, carry=offs)
        def vec_body(v, offs):
            off0, off1 = offs
            vec = idx_buf[pl.ds(v * 16, 16)]
            q = vec - base
            eid = ci * _CH + v * 16 + iota
            m0 = (q >= 0) & (q < _HC)
            pc0 = plsc.all_reduce_population_count(m0)
            inc0 = plsc.cumsum(m0.astype(jnp.int32))
            tgt0 = jnp.where(m0, off0 + inc0 - 1, _CAP + 16)
            plsc.store_scatter(lst, [tgt0], eid | (q << 18))
            off0 = jnp.minimum(off0 + pc0[0], _CAP)
            q1 = q - _HC
            m1 = (q1 >= 0) & (q1 < _HC)
            pc1 = plsc.all_reduce_population_count(m1)
            inc1 = plsc.cumsum(m1.astype(jnp.int32))
            tgt1 = jnp.where(m1, _LS + off1 + inc1 - 1, _CAP + 16)
            plsc.store_scatter(lst, [tgt1], eid | (q1 << 18))
            off1 = jnp.minimum(off1 + pc1[0], _CAP)
            return (off0, off1)

        return vec_body

    n0, n1 = lax.fori_loop(0, _E // _CH, chunk_body,
                           (jnp.int32(0), jnp.int32(0)))

    for p in range(2):
        m_hbm = (m0_hbm, m1_hbm)[p]
        count = p == 0

        def half_body(hh, _):
            lbase = hh * _LS
            cbase = hh * _CS
            n = jnp.where(hh == 0, n0, n1)
            nchunks = (n + _K - 1) // _K

            @plsc.parallel_loop(0, _HC + 1, unroll=2)
            def zr(i):
                for f in range(_FP // 16):
                    accum[i, pl.ds(f * 16, 16)] = z16f

            def fire(ci):
                so = (ci % 2) * _K
                for v in range(_K // 16):
                    pk = lst[pl.ds(lbase + ci * _K + v * 16, 16)]
                    gbuf[pl.ds(so + v * 16, 16)] = pk & 0x3FFFF
                pltpu.make_async_copy(
                    m_hbm.at[gbuf.at[pl.ds(so, _K)]],
                    stage.at[pl.ds(so, _K)], sem).start()

            def wait(ci):
                so = (ci % 2) * _K
                pltpu.make_async_copy(
                    m_hbm.at[gbuf.at[pl.ds(so, _K)]],
                    stage.at[pl.ds(so, _K)], sem).wait()

            def accumulate(ci):
                so = (ci % 2) * _K

                @plsc.parallel_loop(0, _K // 16)
                def grp_body(j16):
                    pkv = lst[pl.ds(lbase + ci * _K + j16 * 16, 16)]
                    qv = pkv >> 18
                    for l in range(16):
                        ql = qv[l]
                        for f in range(_FP // 16):
                            v = stage[so + j16 * 16 + l, pl.ds(f * 16, 16)]
                            plsc.addupdate(accum.at[ql, pl.ds(f * 16, 16)], v)
                        if count:
                            plsc.addupdate(cnt.at[pl.ds(cbase + ql, 16)],
                                           onehot0)

            @pl.when(nchunks > 0)
            def _():
                fire(0)

            def chunk_step(ci, _):
                @pl.when(ci + 1 < nchunks)
                def _():
                    fire(ci + 1)

                wait(ci)
                accumulate(ci)
                return 0

            lax.fori_loop(0, nchunks, chunk_step, 0)

            @plsc.parallel_loop(0, _HC // 16)
            def fin(cc16):
                cntv = cnt[pl.ds(cbase + cc16 * 16, 16)]
                rfv = 1.0 / jnp.maximum(cntv.astype(jnp.float32), 1.0)
                for l in range(16):
                    rf = rfv[l]
                    cc = cc16 * 16 + l
                    for f in range(_FP // 16):
                        accum[cc, pl.ds(f * 16, 16)] = (
                            accum[cc, pl.ds(f * 16, 16)] * rf)

            pltpu.sync_copy(
                accum.at[pl.ds(0, _HC)],
                out_hbm.at[pl.ds(base + hh * _HC, _HC), pl.ds(p * _FP, _FP)])
            return 0

        lax.fori_loop(0, 2, half_body, 0)


def _seg_mean(gidx, m0, m1):
    mesh = plsc.VectorSubcoreMesh(core_axis_name="c", subcore_axis_name="s")
    return pl.kernel(
        _seg_mean_body,
        out_type=jax.ShapeDtypeStruct((_G, _DIM), jnp.float32),
        mesh=mesh,
        compiler_params=pltpu.CompilerParams(needs_layout_passes=False),
        scratch_types=[
            pltpu.VMEM((_CH,), jnp.int32),
            pltpu.VMEM((2 * (_CAP + 32),), jnp.int32),
            pltpu.VMEM((2 * _K,), jnp.int32),
            pltpu.VMEM((2 * _K, _FP), jnp.float32),
            pltpu.VMEM((_HC + 1, _FP), jnp.float32),
            pltpu.VMEM((2 * (_HC + 32),), jnp.int32),
            pltpu.SemaphoreType.DMA,
        ],
    )(gidx, m0, m1)


def _gelu(x):
    return 0.5 * x * (1.0 + lax.erf(x * _INV_SQRT2))


def _sincos(coords, dim=_DIM, ndim=_NDIM, max_wavelength=10000.0):
    ndim_padding = dim % ndim
    dim_per_ndim = (dim - ndim_padding) // ndim
    sincos_padding = dim_per_ndim % 2
    padding = ndim_padding + sincos_padding * ndim
    eff = (dim - padding) // ndim
    half = eff // 2
    omega = 1.0 / (max_wavelength ** (jnp.arange(half, dtype=jnp.float32) / half))
    out = coords[:, :, None].astype(jnp.float32) * omega[None, None, :]
    emb = jnp.concatenate([jnp.sin(out), jnp.cos(out)], axis=-1)
    emb = emb.reshape(coords.shape[0], ndim * eff)
    if padding > 0:
        emb = jnp.pad(emb, ((0, 0), (0, padding)))
    return emb


def _edge_mlp_body(xa_ref, xb_ref, w1a_ref, w1b_ref, b1_ref,
                   w2_ref, b2_ref, w3_ref, b3_ref, o_ref):
    h = (jnp.dot(xa_ref[...], w1a_ref[...], preferred_element_type=jnp.float32)
         + jnp.dot(xb_ref[...], w1b_ref[...], preferred_element_type=jnp.float32)
         + b1_ref[...])
    h = _gelu(h)
    h = _gelu(jnp.dot(h, w2_ref[...], preferred_element_type=jnp.float32)
              + b2_ref[...])
    o_ref[...] = (jnp.dot(h, w3_ref[...], preferred_element_type=jnp.float32)
                  + b3_ref[...])


def _edge_mlp(xa, xb, w1a, w1b, b1, w2, b2, w3, b3, block_e=2048):
    e = xa.shape[0]
    d = _DIM
    grid = (e // block_e,)
    full = lambda shape: pl.BlockSpec(shape, lambda i: (0, 0))
    return pl.pallas_call(
        _edge_mlp_body,
        grid=grid,
        in_specs=[
            pl.BlockSpec((block_e, d), lambda i: (i, 0)),
            pl.BlockSpec((block_e, d), lambda i: (i, 0)),
            full((d, 2 * d)),
            full((d, 2 * d)),
            full((1, 2 * d)),
            full((2 * d, d)),
            full((1, d)),
            full((d, d)),
            full((1, d)),
        ],
        out_specs=pl.BlockSpec((block_e, d), lambda i: (i, 0)),
        out_shape=jax.ShapeDtypeStruct((e, d), jnp.float32),
    )(xa, xb, w1a, w1b, b1.reshape(1, -1), w2, b2.reshape(1, -1),
      w3, b3.reshape(1, -1))


def kernel(mesh_pos, sdf, grid_pos, mesh_to_grid_edges,
           sdf_w1, sdf_b1, sdf_w2, sdf_b2,
           msg_w1, msg_b1, msg_w2, msg_b2, msg_w3, msg_b3):
    g = grid_pos.shape[0]
    mesh_e = _sincos(mesh_pos)
    grid_pe = _sincos(grid_pos)
    s = sdf.reshape(-1, 1)
    s = _gelu(s @ sdf_w1 + sdf_b1) @ sdf_w2 + sdf_b2
    grid_embed = grid_pe + s

    w1a = msg_w1[:_DIM]
    w1b = msg_w1[_DIM:]
    a, b = _precompute(mesh_e, grid_embed, w1a, w1b, msg_b1)

    grid_idx = mesh_to_grid_edges[:, 0]
    mesh_idx = mesh_to_grid_edges[:, 1]

    xa = jnp.take(a, mesh_idx, axis=0)
    xb = jnp.take(b, grid_idx, axis=0)

    m0, m1 = _edge_mlp_pre(xa, xb, msg_w2, msg_b2, msg_w3, msg_b3)

    mean = _seg_mean(grid_idx, m0, m1)
    return mean.reshape(1, g, _DIM)


def _edge_mlp_pre_body(xa_ref, xb_ref, w2_ref, b2_ref,
                       w3_ref, b3_ref, o0_ref, o1_ref):
    alo, ahi = _unpack32(xa_ref[...])
    blo, bhi = _unpack32(xb_ref[...])
    h = _gelu(jnp.concatenate([alo + blo, ahi + bhi], axis=1))
    h = _gelu(jnp.dot(h.astype(jnp.bfloat16), w2_ref[...],
                      preferred_element_type=jnp.float32) + b2_ref[...])
    o = (jnp.dot(h.astype(jnp.bfloat16), w3_ref[...],
                 preferred_element_type=jnp.float32) + b3_ref[...])
    o0_ref[...] = o[:, :_FP]
    o1_ref[...] = o[:, _FP:]


def _edge_mlp_pre(xa, xb, w2, b2, w3, b3, block_e=2048):
    e = xa.shape[0]
    d = _DIM
    full = lambda shape: pl.BlockSpec(shape, lambda i: (0, 0))
    return pl.pallas_call(
        _edge_mlp_pre_body,
        grid=(e // block_e,),
        in_specs=[
            pl.BlockSpec((block_e, d), lambda i: (i, 0)),
            pl.BlockSpec((block_e, d), lambda i: (i, 0)),
            full((2 * d, d)),
            full((1, d)),
            full((d, d)),
            full((1, d)),
        ],
        out_specs=[pl.BlockSpec((block_e, _FP), lambda i: (i, 0)),
                   pl.BlockSpec((block_e, _FP), lambda i: (i, 0))],
        out_shape=[jax.ShapeDtypeStruct((e, _FP), jnp.float32),
                   jax.ShapeDtypeStruct((e, _FP), jnp.float32)],
    )(xa, xb, w2.astype(jnp.bfloat16), b2.reshape(1, -1),
      w3.astype(jnp.bfloat16), b3.reshape(1, -1))


def _pack32(x):
    lo = lax.bitcast_convert_type(x[:, :_DIM], jnp.int16).astype(jnp.int32)
    hi = lax.bitcast_convert_type(x[:, _DIM:], jnp.int16).astype(jnp.int32)
    return lax.bitcast_convert_type((lo & 0xFFFF) | (hi << 16), jnp.float32)


def _unpack32(wf):
    w = lax.bitcast_convert_type(wf, jnp.int32)
    lo = lax.bitcast_convert_type(w.astype(jnp.int16), jnp.bfloat16)
    hi = lax.bitcast_convert_type((w >> 16).astype(jnp.int16), jnp.bfloat16)
    return lo.astype(jnp.float32), hi.astype(jnp.float32)


def _precompute_body(me_ref, ge_ref, w1a_ref, w1b_ref, b1_ref, a_ref, b_ref):
    a = jnp.dot(me_ref[...].astype(jnp.bfloat16), w1a_ref[...],
                preferred_element_type=jnp.float32).astype(jnp.bfloat16)
    a_ref[...] = _pack32(a)
    b = (jnp.dot(ge_ref[...].astype(jnp.bfloat16), w1b_ref[...],
                 preferred_element_type=jnp.float32)
         + b1_ref[...]).astype(jnp.bfloat16)
    b_ref[...] = _pack32(b)


def _precompute(mesh_e, grid_embed, w1a, w1b, b1, block_n=2048):
    n = mesh_e.shape[0]
    d = _DIM
    full = lambda shape: pl.BlockSpec(shape, lambda i: (0, 0))
    return pl.pallas_call(
        _precompute_body,
        grid=(n // block_n,),
        in_specs=[
            pl.BlockSpec((block_n, d), lambda i: (i, 0)),
            pl.BlockSpec((block_n, d), lambda i: (i, 0)),
            full((d, 2 * d)),
            full((d, 2 * d)),
            full((1, 2 * d)),
        ],
        out_specs=[pl.BlockSpec((block_n, d), lambda i: (i, 0)),
                   pl.BlockSpec((block_n, d), lambda i: (i, 0))],
        out_shape=[jax.ShapeDtypeStruct((n, d), jnp.float32),
                   jax.ShapeDtypeStruct((n, d), jnp.float32)],
    )(mesh_e, grid_embed, w1a.astype(jnp.bfloat16), w1b.astype(jnp.bfloat16),
      b1.reshape(1, -1))


# speedup vs baseline: 6.7497x; 2.0931x over previous
import functools

import jax
import jax.numpy as jnp
from jax import lax
from jax.experimental import pallas as pl
from jax.experimental.pallas import tpu as pltpu
from jax.experimental.pallas import tpu_sc as plsc

_DIM = 256
_NDIM = 3
_INV_SQRT2 = 0.7071067811865476

_G = 32768
_E = 262144
_NW = 32
_CPT = _G // _NW
_HC = _CPT // 2
_FP = 128
_CAP = 6144
_CH = 8192
_K = 128


def _seg_mean_body(gidx_hbm, m0_hbm, m1_hbm, out_hbm, idx_buf, lst,
                   gbuf, stage, accum, cnt, sem):
    c = lax.axis_index("c")
    s = lax.axis_index("s")
    wid = s * 2 + c
    base = wid * _CPT
    z16f = jnp.zeros((16,), jnp.float32)
    z16i = jnp.zeros((16,), jnp.int32)
    pad16 = jnp.full((16,), _HC << 18, jnp.int32)
    iota = lax.iota(jnp.int32, 16)
    onehot0 = jnp.where(iota == 0, 1, 0).astype(jnp.int32)
    _LS = _CAP + 32
    _CS = _HC + 32

    @plsc.parallel_loop(0, (2 * _LS) // 16, unroll=2)
    def pre(i):
        lst[pl.ds(i * 16, 16)] = pad16

    @plsc.parallel_loop(0, (2 * _CS) // 16)
    def zc(i):
        cnt[pl.ds(i * 16, 16)] = z16i

    def chunk_body(ci, offs):
        pltpu.sync_copy(gidx_hbm.at[pl.ds(ci * _CH, _CH)], idx_buf)

        @plsc.parallel_loop(0, _CH // 16, carry=offs)
        def vec_body(v, offs):
            off0, off1 = offs
            vec = idx_buf[pl.ds(v * 16, 16)]
            q = vec - base
            eid = ci * _CH + v * 16 + iota
            m0 = (q >= 0) & (q < _HC)
            pc0 = plsc.all_reduce_population_count(m0)
            inc0 = plsc.cumsum(m0.astype(jnp.int32))
            tgt0 = jnp.where(m0, off0 + inc0 - 1, _CAP + 16)
            plsc.store_scatter(lst, [tgt0], eid | (q << 18))
            off0 = jnp.minimum(off0 + pc0[0], _CAP)
            q1 = q - _HC
            m1 = (q1 >= 0) & (q1 < _HC)
            pc1 = plsc.all_reduce_population_count(m1)
            inc1 = plsc.cumsum(m1.astype(jnp.int32))
            tgt1 = jnp.where(m1, _LS + off1 + inc1 - 1, _CAP + 16)
            plsc.store_scatter(lst, [tgt1], eid | (q1 << 18))
            off1 = jnp.minimum(off1 + pc1[0], _CAP)
            return (off0, off1)

        return vec_body

    n0, n1 = lax.fori_loop(0, _E // _CH, chunk_body,
                           (jnp.int32(0), jnp.int32(0)))

    for p in range(2):
        m_hbm = (m0_hbm, m1_hbm)[p]
        count = p == 0

        def half_body(hh, _):
            lbase = hh * _LS
            cbase = hh * _CS
            n = jnp.where(hh == 0, n0, n1)
            nchunks = (n + _K - 1) // _K

            @plsc.parallel_loop(0, _HC + 1, unroll=2)
            def zr(i):
                for f in range(_FP // 16):
                    accum[i, pl.ds(f * 16, 16)] = z16f

            def fire(ci):
                so = (ci % 2) * _K
                for v in range(_K // 16):
                    pk = lst[pl.ds(lbase + ci * _K + v * 16, 16)]
                    gbuf[pl.ds(so + v * 16, 16)] = pk & 0x3FFFF
                pltpu.make_async_copy(
                    m_hbm.at[gbuf.at[pl.ds(so, _K)]],
                    stage.at[pl.ds(so, _K)], sem).start()

            def wait(ci):
                so = (ci % 2) * _K
                pltpu.make_async_copy(
                    m_hbm.at[gbuf.at[pl.ds(so, _K)]],
                    stage.at[pl.ds(so, _K)], sem).wait()

            def accumulate(ci):
                so = (ci % 2) * _K

                @plsc.parallel_loop(0, _K // 16)
                def grp_body(j16):
                    pkv = lst[pl.ds(lbase + ci * _K + j16 * 16, 16)]
                    qv = pkv >> 18
                    for l in range(16):
                        ql = qv[l]
                        for f in range(_FP // 16):
                            v = stage[so + j16 * 16 + l, pl.ds(f * 16, 16)]
                            plsc.addupdate(accum.at[ql, pl.ds(f * 16, 16)], v)
                        if count:
                            plsc.addupdate(cnt.at[pl.ds(cbase + ql, 16)],
                                           onehot0)

            @pl.when(nchunks > 0)
            def _():
                fire(0)

            def chunk_step(ci, _):
                @pl.when(ci + 1 < nchunks)
                def _():
                    fire(ci + 1)

                wait(ci)
                accumulate(ci)
                return 0

            lax.fori_loop(0, nchunks, chunk_step, 0)

            @plsc.parallel_loop(0, _HC // 16)
            def fin(cc16):
                cntv = cnt[pl.ds(cbase + cc16 * 16, 16)]
                rfv = 1.0 / jnp.maximum(cntv.astype(jnp.float32), 1.0)
                for l in range(16):
                    rf = rfv[l]
                    cc = cc16 * 16 + l
                    for f in range(_FP // 16):
                        accum[cc, pl.ds(f * 16, 16)] = (
                            accum[cc, pl.ds(f * 16, 16)] * rf)

            pltpu.sync_copy(
                accum.at[pl.ds(0, _HC)],
                out_hbm.at[pl.ds(base + hh * _HC, _HC), pl.ds(p * _FP, _FP)])
            return 0

        lax.fori_loop(0, 2, half_body, 0)


_GK = 128
_EPT = _E // _NW


def _gather_body(idxa_hbm, idxb_hbm, a_hbm, b_hbm, xa_hbm, xb_hbm,
                 idx_buf, stage, sem_g, sem_s):
    c = lax.axis_index("c")
    s = lax.axis_index("s")
    wid = s * 2 + c
    base = wid * _EPT
    nc = _EPT // _GK

    for idx_hbm, src, dst in ((idxa_hbm, a_hbm, xa_hbm),
                              (idxb_hbm, b_hbm, xb_hbm)):
        pltpu.sync_copy(idx_hbm.at[pl.ds(base, _EPT)], idx_buf)

        def fire_g(ci):
            so = (ci % 2) * _GK
            pltpu.make_async_copy(
                src.at[idx_buf.at[pl.ds(ci * _GK, _GK)]],
                stage.at[pl.ds(so, _GK)], sem_g).start()

        def wait_g(ci):
            so = (ci % 2) * _GK
            pltpu.make_async_copy(
                src.at[idx_buf.at[pl.ds(ci * _GK, _GK)]],
                stage.at[pl.ds(so, _GK)], sem_g).wait()

        def fire_s(ci):
            so = (ci % 2) * _GK
            pltpu.make_async_copy(
                stage.at[pl.ds(so, _GK)],
                dst.at[pl.ds(base + ci * _GK, _GK)], sem_s).start()

        def wait_s(ci):
            so = (ci % 2) * _GK
            pltpu.make_async_copy(
                stage.at[pl.ds(so, _GK)],
                dst.at[pl.ds(base + ci * _GK, _GK)], sem_s).wait()

        fire_g(0)

        def step(ci, _):
            wait_g(ci)
            fire_s(ci)

            @pl.when(ci + 1 < nc)
            def _():
                @pl.when(ci >= 1)
                def _():
                    wait_s(ci - 1)
                fire_g(ci + 1)
            return 0

        lax.fori_loop(0, nc, step, 0)
        wait_s(nc - 2)
        wait_s(nc - 1)


def _sc_gather(mesh_idx, grid_idx, a, b):
    mesh = plsc.VectorSubcoreMesh(core_axis_name="c", subcore_axis_name="s")
    return pl.kernel(
        _gather_body,
        out_type=[jax.ShapeDtypeStruct((_E, _DIM), jnp.float32),
                  jax.ShapeDtypeStruct((_E, _DIM), jnp.float32)],
        mesh=mesh,
        compiler_params=pltpu.CompilerParams(needs_layout_passes=False),
        scratch_types=[
            pltpu.VMEM((_EPT,), jnp.int32),
            pltpu.VMEM((2 * _GK, _DIM), jnp.float32),
            pltpu.SemaphoreType.DMA,
            pltpu.SemaphoreType.DMA,
        ],
    )(mesh_idx, grid_idx, a, b)


def _seg_mean(gidx, m0, m1):
    mesh = plsc.VectorSubcoreMesh(core_axis_name="c", subcore_axis_name="s")
    return pl.kernel(
        _seg_mean_body,
        out_type=jax.ShapeDtypeStruct((_G, _DIM), jnp.float32),
        mesh=mesh,
        compiler_params=pltpu.CompilerParams(needs_layout_passes=False),
        scratch_types=[
            pltpu.VMEM((_CH,), jnp.int32),
            pltpu.VMEM((2 * (_CAP + 32),), jnp.int32),
            pltpu.VMEM((2 * _K,), jnp.int32),
            pltpu.VMEM((2 * _K, _FP), jnp.float32),
            pltpu.VMEM((_HC + 1, _FP), jnp.float32),
            pltpu.VMEM((2 * (_HC + 32),), jnp.int32),
            pltpu.SemaphoreType.DMA,
        ],
    )(gidx, m0, m1)


def _gelu(x):
    return 0.5 * x * (1.0 + lax.erf(x * _INV_SQRT2))


def _sincos(coords, dim=_DIM, ndim=_NDIM, max_wavelength=10000.0):
    ndim_padding = dim % ndim
    dim_per_ndim = (dim - ndim_padding) // ndim
    sincos_padding = dim_per_ndim % 2
    padding = ndim_padding + sincos_padding * ndim
    eff = (dim - padding) // ndim
    half = eff // 2
    omega = 1.0 / (max_wavelength ** (jnp.arange(half, dtype=jnp.float32) / half))
    out = coords[:, :, None].astype(jnp.float32) * omega[None, None, :]
    emb = jnp.concatenate([jnp.sin(out), jnp.cos(out)], axis=-1)
    emb = emb.reshape(coords.shape[0], ndim * eff)
    if padding > 0:
        emb = jnp.pad(emb, ((0, 0), (0, padding)))
    return emb


def _edge_mlp_body(xa_ref, xb_ref, w1a_ref, w1b_ref, b1_ref,
                   w2_ref, b2_ref, w3_ref, b3_ref, o_ref):
    h = (jnp.dot(xa_ref[...], w1a_ref[...], preferred_element_type=jnp.float32)
         + jnp.dot(xb_ref[...], w1b_ref[...], preferred_element_type=jnp.float32)
         + b1_ref[...])
    h = _gelu(h)
    h = _gelu(jnp.dot(h, w2_ref[...], preferred_element_type=jnp.float32)
              + b2_ref[...])
    o_ref[...] = (jnp.dot(h, w3_ref[...], preferred_element_type=jnp.float32)
                  + b3_ref[...])


def _edge_mlp(xa, xb, w1a, w1b, b1, w2, b2, w3, b3, block_e=2048):
    e = xa.shape[0]
    d = _DIM
    grid = (e // block_e,)
    full = lambda shape: pl.BlockSpec(shape, lambda i: (0, 0))
    return pl.pallas_call(
        _edge_mlp_body,
        grid=grid,
        in_specs=[
            pl.BlockSpec((block_e, d), lambda i: (i, 0)),
            pl.BlockSpec((block_e, d), lambda i: (i, 0)),
            full((d, 2 * d)),
            full((d, 2 * d)),
            full((1, 2 * d)),
            full((2 * d, d)),
            full((1, d)),
            full((d, d)),
            full((1, d)),
        ],
        out_specs=pl.BlockSpec((block_e, d), lambda i: (i, 0)),
        out_shape=jax.ShapeDtypeStruct((e, d), jnp.float32),
    )(xa, xb, w1a, w1b, b1.reshape(1, -1), w2, b2.reshape(1, -1),
      w3, b3.reshape(1, -1))


def kernel(mesh_pos, sdf, grid_pos, mesh_to_grid_edges,
           sdf_w1, sdf_b1, sdf_w2, sdf_b2,
           msg_w1, msg_b1, msg_w2, msg_b2, msg_w3, msg_b3):
    g = grid_pos.shape[0]
    mesh_e = _sincos(mesh_pos)
    grid_pe = _sincos(grid_pos)
    s = sdf.reshape(-1, 1)
    s = _gelu(s @ sdf_w1 + sdf_b1) @ sdf_w2 + sdf_b2
    grid_embed = grid_pe + s

    w1a = msg_w1[:_DIM]
    w1b = msg_w1[_DIM:]
    a, b = _precompute(mesh_e, grid_embed, w1a, w1b, msg_b1)

    grid_idx = mesh_to_grid_edges[:, 0]
    mesh_idx = mesh_to_grid_edges[:, 1]

    xa, xb = _sc_gather(mesh_idx, grid_idx, a, b)

    m0, m1 = _edge_mlp_pre(xa, xb, msg_w2, msg_b2, msg_w3, msg_b3)

    mean = _seg_mean(grid_idx, m0, m1)
    return mean.reshape(1, g, _DIM)


def _edge_mlp_pre_body(xa_ref, xb_ref, w2_ref, b2_ref,
                       w3_ref, b3_ref, o0_ref, o1_ref):
    alo, ahi = _unpack32(xa_ref[...])
    blo, bhi = _unpack32(xb_ref[...])
    h = _gelu(jnp.concatenate([alo + blo, ahi + bhi], axis=1))
    h = _gelu(jnp.dot(h.astype(jnp.bfloat16), w2_ref[...],
                      preferred_element_type=jnp.float32) + b2_ref[...])
    o = (jnp.dot(h.astype(jnp.bfloat16), w3_ref[...],
                 preferred_element_type=jnp.float32) + b3_ref[...])
    o0_ref[...] = o[:, :_FP]
    o1_ref[...] = o[:, _FP:]


def _edge_mlp_pre(xa, xb, w2, b2, w3, b3, block_e=2048):
    e = xa.shape[0]
    d = _DIM
    full = lambda shape: pl.BlockSpec(shape, lambda i: (0, 0))
    return pl.pallas_call(
        _edge_mlp_pre_body,
        grid=(e // block_e,),
        in_specs=[
            pl.BlockSpec((block_e, d), lambda i: (i, 0)),
            pl.BlockSpec((block_e, d), lambda i: (i, 0)),
            full((2 * d, d)),
            full((1, d)),
            full((d, d)),
            full((1, d)),
        ],
        out_specs=[pl.BlockSpec((block_e, _FP), lambda i: (i, 0)),
                   pl.BlockSpec((block_e, _FP), lambda i: (i, 0))],
        out_shape=[jax.ShapeDtypeStruct((e, _FP), jnp.float32),
                   jax.ShapeDtypeStruct((e, _FP), jnp.float32)],
    )(xa, xb, w2.astype(jnp.bfloat16), b2.reshape(1, -1),
      w3.astype(jnp.bfloat16), b3.reshape(1, -1))


def _pack32(x):
    lo = lax.bitcast_convert_type(x[:, :_DIM], jnp.int16).astype(jnp.int32)
    hi = lax.bitcast_convert_type(x[:, _DIM:], jnp.int16).astype(jnp.int32)
    return lax.bitcast_convert_type((lo & 0xFFFF) | (hi << 16), jnp.float32)


def _unpack32(wf):
    w = lax.bitcast_convert_type(wf, jnp.int32)
    lo = lax.bitcast_convert_type(w.astype(jnp.int16), jnp.bfloat16)
    hi = lax.bitcast_convert_type((w >> 16).astype(jnp.int16), jnp.bfloat16)
    return lo.astype(jnp.float32), hi.astype(jnp.float32)


def _precompute_body(me_ref, ge_ref, w1a_ref, w1b_ref, b1_ref, a_ref, b_ref):
    a = jnp.dot(me_ref[...].astype(jnp.bfloat16), w1a_ref[...],
                preferred_element_type=jnp.float32).astype(jnp.bfloat16)
    a_ref[...] = _pack32(a)
    b = (jnp.dot(ge_ref[...].astype(jnp.bfloat16), w1b_ref[...],
                 preferred_element_type=jnp.float32)
         + b1_ref[...]).astype(jnp.bfloat16)
    b_ref[...] = _pack32(b)


def _precompute(mesh_e, grid_embed, w1a, w1b, b1, block_n=2048):
    n = mesh_e.shape[0]
    d = _DIM
    full = lambda shape: pl.BlockSpec(shape, lambda i: (0, 0))
    return pl.pallas_call(
        _precompute_body,
        grid=(n // block_n,),
        in_specs=[
            pl.BlockSpec((block_n, d), lambda i: (i, 0)),
            pl.BlockSpec((block_n, d), lambda i: (i, 0)),
            full((d, 2 * d)),
            full((d, 2 * d)),
            full((1, 2 * d)),
        ],
        out_specs=[pl.BlockSpec((block_n, d), lambda i: (i, 0)),
                   pl.BlockSpec((block_n, d), lambda i: (i, 0))],
        out_shape=[jax.ShapeDtypeStruct((n, d), jnp.float32),
                   jax.ShapeDtypeStruct((n, d), jnp.float32)],
    )(mesh_e, grid_embed, w1a.astype(jnp.bfloat16), w1b.astype(jnp.bfloat16),
      b1.reshape(1, -1))


# speedup vs baseline: 6.8158x; 1.0098x over previous
import functools

import jax
import jax.numpy as jnp
from jax import lax
from jax.experimental import pallas as pl
from jax.experimental.pallas import tpu as pltpu
from jax.experimental.pallas import tpu_sc as plsc

_DIM = 256
_NDIM = 3
_INV_SQRT2 = 0.7071067811865476

_G = 32768
_E = 262144
_NW = 32
_CPT = _G // _NW
_HC = _CPT // 2
_FP = 128
_CAP = 6144
_CAPT = 9728
_CH = 8192
_K = 128


def _seg_mean_body(gidx_hbm, m0_hbm, m1_hbm, out_hbm, idx_buf, lst,
                   gbuf, stage, accum, cnt, sem):
    c = lax.axis_index("c")
    s = lax.axis_index("s")
    wid = s * 2 + c
    base = wid * _CPT
    z16f = jnp.zeros((16,), jnp.float32)
    z16i = jnp.zeros((16,), jnp.int32)
    pad16 = jnp.full((16,), _HC << 18, jnp.int32)
    iota = lax.iota(jnp.int32, 16)
    onehot0 = jnp.where(iota == 0, 1, 0).astype(jnp.int32)
    _LS = _CAP + 32
    _CS = _HC + 32

    @plsc.parallel_loop(0, (2 * _LS + _CAPT + 32) // 16, unroll=2)
    def pre(i):
        lst[pl.ds(i * 16, 16)] = pad16

    @plsc.parallel_loop(0, (2 * _CS) // 16)
    def zc(i):
        cnt[pl.ds(i * 16, 16)] = z16i

    def chunk_body(ci, offs):
        pltpu.sync_copy(gidx_hbm.at[pl.ds(ci * _CH, _CH)], idx_buf)

        @plsc.parallel_loop(0, _CH // 16, carry=offs)
        def vec_body(v, offt):
            vec = idx_buf[pl.ds(v * 16, 16)]
            q = vec - base
            eid = ci * _CH + v * 16 + iota
            mt = (q >= 0) & (q < _CPT)
            pct = plsc.all_reduce_population_count(mt)
            inct = plsc.cumsum(mt.astype(jnp.int32))
            tgtt = jnp.where(mt, offt + inct - 1, 2 * _LS + _CAPT + 16)
            plsc.store_scatter(lst, [2 * _LS + tgtt], eid | (q << 18))
            return jnp.minimum(offt + pct[0], _CAPT)

        return vec_body

    nt = lax.fori_loop(0, _E // _CH, chunk_body, jnp.int32(0))

    def split_body(v, offs):
        off0, off1 = offs
        pk = lst[pl.ds(2 * _LS + v * 16, 16)]
        valid = (v * 16 + iota) < nt
        q = pk >> 18
        m0 = valid & (q < _HC)
        pc0 = plsc.all_reduce_population_count(m0)
        inc0 = plsc.cumsum(m0.astype(jnp.int32))
        tgt0 = jnp.where(m0, off0 + inc0 - 1, _CAP + 16)
        plsc.store_scatter(lst, [tgt0], pk)
        off0 = jnp.minimum(off0 + pc0[0], _CAP)
        m1 = valid & (q >= _HC)
        pc1 = plsc.all_reduce_population_count(m1)
        inc1 = plsc.cumsum(m1.astype(jnp.int32))
        tgt1 = jnp.where(m1, _LS + off1 + inc1 - 1, _CAP + 16)
        plsc.store_scatter(lst, [tgt1], pk - (_HC << 18))
        off1 = jnp.minimum(off1 + pc1[0], _CAP)
        return (off0, off1)

    n0, n1 = lax.fori_loop(0, (_CAPT + 16) // 16, split_body,
                           (jnp.int32(0), jnp.int32(0)))

    for p in range(2):
        m_hbm = (m0_hbm, m1_hbm)[p]
        count = p == 0

        def half_body(hh, _):
            lbase = hh * _LS
            cbase = hh * _CS
            n = jnp.where(hh == 0, n0, n1)
            nchunks = (n + _K - 1) // _K

            @plsc.parallel_loop(0, _HC + 1, unroll=2)
            def zr(i):
                for f in range(_FP // 16):
                    accum[i, pl.ds(f * 16, 16)] = z16f

            def fire(ci):
                so = (ci % 2) * _K
                for v in range(_K // 16):
                    pk = lst[pl.ds(lbase + ci * _K + v * 16, 16)]
                    gbuf[pl.ds(so + v * 16, 16)] = pk & 0x3FFFF
                pltpu.make_async_copy(
                    m_hbm.at[gbuf.at[pl.ds(so, _K)]],
                    stage.at[pl.ds(so, _K)], sem).start()

            def wait(ci):
                so = (ci % 2) * _K
                pltpu.make_async_copy(
                    m_hbm.at[gbuf.at[pl.ds(so, _K)]],
                    stage.at[pl.ds(so, _K)], sem).wait()

            def accumulate(ci):
                so = (ci % 2) * _K

                @plsc.parallel_loop(0, _K // 16)
                def grp_body(j16):
                    pkv = lst[pl.ds(lbase + ci * _K + j16 * 16, 16)]
                    qv = pkv >> 18
                    for l in range(16):
                        ql = qv[l]
                        for f in range(_FP // 16):
                            v = stage[so + j16 * 16 + l, pl.ds(f * 16, 16)]
                            plsc.addupdate(accum.at[ql, pl.ds(f * 16, 16)], v)
                        if count:
                            plsc.addupdate(cnt.at[pl.ds(cbase + ql, 16)],
                                           onehot0)

            @pl.when(nchunks > 0)
            def _():
                fire(0)

            def chunk_step(ci, _):
                @pl.when(ci + 1 < nchunks)
                def _():
                    fire(ci + 1)

                wait(ci)
                accumulate(ci)
                return 0

            lax.fori_loop(0, nchunks, chunk_step, 0)

            @plsc.parallel_loop(0, _HC // 16)
            def fin(cc16):
                cntv = cnt[pl.ds(cbase + cc16 * 16, 16)]
                rfv = 1.0 / jnp.maximum(cntv.astype(jnp.float32), 1.0)
                for l in range(16):
                    rf = rfv[l]
                    cc = cc16 * 16 + l
                    for f in range(_FP // 16):
                        accum[cc, pl.ds(f * 16, 16)] = (
                            accum[cc, pl.ds(f * 16, 16)] * rf)

            pltpu.sync_copy(
                accum.at[pl.ds(0, _HC)],
                out_hbm.at[pl.ds(base + hh * _HC, _HC), pl.ds(p * _FP, _FP)])
            return 0

        lax.fori_loop(0, 2, half_body, 0)


_GK = 128
_EPT = _E // _NW


def _gather_body(idxa_hbm, idxb_hbm, a_hbm, b_hbm, xa_hbm, xb_hbm,
                 idx_buf, stage, sem_g, sem_s):
    c = lax.axis_index("c")
    s = lax.axis_index("s")
    wid = s * 2 + c
    base = wid * _EPT
    nc = _EPT // _GK

    for idx_hbm, src, dst in ((idxa_hbm, a_hbm, xa_hbm),
                              (idxb_hbm, b_hbm, xb_hbm)):
        pltpu.sync_copy(idx_hbm.at[pl.ds(base, _EPT)], idx_buf)

        def fire_g(ci):
            so = (ci % 2) * _GK
            pltpu.make_async_copy(
                src.at[idx_buf.at[pl.ds(ci * _GK, _GK)]],
                stage.at[pl.ds(so, _GK)], sem_g).start()

        def wait_g(ci):
            so = (ci % 2) * _GK
            pltpu.make_async_copy(
                src.at[idx_buf.at[pl.ds(ci * _GK, _GK)]],
                stage.at[pl.ds(so, _GK)], sem_g).wait()

        def fire_s(ci):
            so = (ci % 2) * _GK
            pltpu.make_async_copy(
                stage.at[pl.ds(so, _GK)],
                dst.at[pl.ds(base + ci * _GK, _GK)], sem_s).start()

        def wait_s(ci):
            so = (ci % 2) * _GK
            pltpu.make_async_copy(
                stage.at[pl.ds(so, _GK)],
                dst.at[pl.ds(base + ci * _GK, _GK)], sem_s).wait()

        fire_g(0)

        def step(ci, _):
            wait_g(ci)
            fire_s(ci)

            @pl.when(ci + 1 < nc)
            def _():
                @pl.when(ci >= 1)
                def _():
                    wait_s(ci - 1)
                fire_g(ci + 1)
            return 0

        lax.fori_loop(0, nc, step, 0)
        wait_s(nc - 2)
        wait_s(nc - 1)


def _sc_gather(mesh_idx, grid_idx, a, b):
    mesh = plsc.VectorSubcoreMesh(core_axis_name="c", subcore_axis_name="s")
    return pl.kernel(
        _gather_body,
        out_type=[jax.ShapeDtypeStruct((_E, _DIM), jnp.float32),
                  jax.ShapeDtypeStruct((_E, _DIM), jnp.float32)],
        mesh=mesh,
        compiler_params=pltpu.CompilerParams(needs_layout_passes=False),
        scratch_types=[
            pltpu.VMEM((_EPT,), jnp.int32),
            pltpu.VMEM((2 * _GK, _DIM), jnp.float32),
            pltpu.SemaphoreType.DMA,
            pltpu.SemaphoreType.DMA,
        ],
    )(mesh_idx, grid_idx, a, b)


def _seg_mean(gidx, m0, m1):
    mesh = plsc.VectorSubcoreMesh(core_axis_name="c", subcore_axis_name="s")
    return pl.kernel(
        _seg_mean_body,
        out_type=jax.ShapeDtypeStruct((_G, _DIM), jnp.float32),
        mesh=mesh,
        compiler_params=pltpu.CompilerParams(needs_layout_passes=False),
        scratch_types=[
            pltpu.VMEM((_CH,), jnp.int32),
            pltpu.VMEM((2 * (_CAP + 32) + _CAPT + 32,), jnp.int32),
            pltpu.VMEM((2 * _K,), jnp.int32),
            pltpu.VMEM((2 * _K, _FP), jnp.float32),
            pltpu.VMEM((_HC + 1, _FP), jnp.float32),
            pltpu.VMEM((2 * (_HC + 32),), jnp.int32),
            pltpu.SemaphoreType.DMA,
        ],
    )(gidx, m0, m1)


def _gelu(x):
    return 0.5 * x * (1.0 + lax.erf(x * _INV_SQRT2))


def _sincos(coords, dim=_DIM, ndim=_NDIM, max_wavelength=10000.0):
    ndim_padding = dim % ndim
    dim_per_ndim = (dim - ndim_padding) // ndim
    sincos_padding = dim_per_ndim % 2
    padding = ndim_padding + sincos_padding * ndim
    eff = (dim - padding) // ndim
    half = eff // 2
    omega = 1.0 / (max_wavelength ** (jnp.arange(half, dtype=jnp.float32) / half))
    out = coords[:, :, None].astype(jnp.float32) * omega[None, None, :]
    emb = jnp.concatenate([jnp.sin(out), jnp.cos(out)], axis=-1)
    emb = emb.reshape(coords.shape[0], ndim * eff)
    if padding > 0:
        emb = jnp.pad(emb, ((0, 0), (0, padding)))
    return emb


def _edge_mlp_body(xa_ref, xb_ref, w1a_ref, w1b_ref, b1_ref,
                   w2_ref, b2_ref, w3_ref, b3_ref, o_ref):
    h = (jnp.dot(xa_ref[...], w1a_ref[...], preferred_element_type=jnp.float32)
         + jnp.dot(xb_ref[...], w1b_ref[...], preferred_element_type=jnp.float32)
         + b1_ref[...])
    h = _gelu(h)
    h = _gelu(jnp.dot(h, w2_ref[...], preferred_element_type=jnp.float32)
              + b2_ref[...])
    o_ref[...] = (jnp.dot(h, w3_ref[...], preferred_element_type=jnp.float32)
                  + b3_ref[...])


def _edge_mlp(xa, xb, w1a, w1b, b1, w2, b2, w3, b3, block_e=2048):
    e = xa.shape[0]
    d = _DIM
    grid = (e // block_e,)
    full = lambda shape: pl.BlockSpec(shape, lambda i: (0, 0))
    return pl.pallas_call(
        _edge_mlp_body,
        grid=grid,
        in_specs=[
            pl.BlockSpec((block_e, d), lambda i: (i, 0)),
            pl.BlockSpec((block_e, d), lambda i: (i, 0)),
            full((d, 2 * d)),
            full((d, 2 * d)),
            full((1, 2 * d)),
            full((2 * d, d)),
            full((1, d)),
            full((d, d)),
            full((1, d)),
        ],
        out_specs=pl.BlockSpec((block_e, d), lambda i: (i, 0)),
        out_shape=jax.ShapeDtypeStruct((e, d), jnp.float32),
    )(xa, xb, w1a, w1b, b1.reshape(1, -1), w2, b2.reshape(1, -1),
      w3, b3.reshape(1, -1))


def kernel(mesh_pos, sdf, grid_pos, mesh_to_grid_edges,
           sdf_w1, sdf_b1, sdf_w2, sdf_b2,
           msg_w1, msg_b1, msg_w2, msg_b2, msg_w3, msg_b3):
    g = grid_pos.shape[0]
    mesh_e = _sincos(mesh_pos)
    grid_pe = _sincos(grid_pos)
    s = sdf.reshape(-1, 1)
    s = _gelu(s @ sdf_w1 + sdf_b1) @ sdf_w2 + sdf_b2
    grid_embed = grid_pe + s

    w1a = msg_w1[:_DIM]
    w1b = msg_w1[_DIM:]
    a, b = _precompute(mesh_e, grid_embed, w1a, w1b, msg_b1)

    grid_idx = mesh_to_grid_edges[:, 0]
    mesh_idx = mesh_to_grid_edges[:, 1]

    xa, xb = _sc_gather(mesh_idx, grid_idx, a, b)

    m0, m1 = _edge_mlp_pre(xa, xb, msg_w2, msg_b2, msg_w3, msg_b3)

    mean = _seg_mean(grid_idx, m0, m1)
    return mean.reshape(1, g, _DIM)


def _edge_mlp_pre_body(xa_ref, xb_ref, w2_ref, b2_ref,
                       w3_ref, b3_ref, o0_ref, o1_ref):
    alo, ahi = _unpack32(xa_ref[...])
    blo, bhi = _unpack32(xb_ref[...])
    h = _gelu(jnp.concatenate([alo + blo, ahi + bhi], axis=1))
    h = _gelu(jnp.dot(h.astype(jnp.bfloat16), w2_ref[...],
                      preferred_element_type=jnp.float32) + b2_ref[...])
    o = (jnp.dot(h.astype(jnp.bfloat16), w3_ref[...],
                 preferred_element_type=jnp.float32) + b3_ref[...])
    o0_ref[...] = o[:, :_FP]
    o1_ref[...] = o[:, _FP:]


def _edge_mlp_pre(xa, xb, w2, b2, w3, b3, block_e=2048):
    e = xa.shape[0]
    d = _DIM
    full = lambda shape: pl.BlockSpec(shape, lambda i: (0, 0))
    return pl.pallas_call(
        _edge_mlp_pre_body,
        grid=(e // block_e,),
        in_specs=[
            pl.BlockSpec((block_e, d), lambda i: (i, 0)),
            pl.BlockSpec((block_e, d), lambda i: (i, 0)),
            full((2 * d, d)),
            full((1, d)),
            full((d, d)),
            full((1, d)),
        ],
        out_specs=[pl.BlockSpec((block_e, _FP), lambda i: (i, 0)),
                   pl.BlockSpec((block_e, _FP), lambda i: (i, 0))],
        out_shape=[jax.ShapeDtypeStruct((e, _FP), jnp.float32),
                   jax.ShapeDtypeStruct((e, _FP), jnp.float32)],
    )(xa, xb, w2.astype(jnp.bfloat16), b2.reshape(1, -1),
      w3.astype(jnp.bfloat16), b3.reshape(1, -1))


def _pack32(x):
    lo = lax.bitcast_convert_type(x[:, :_DIM], jnp.int16).astype(jnp.int32)
    hi = lax.bitcast_convert_type(x[:, _DIM:], jnp.int16).astype(jnp.int32)
    return lax.bitcast_convert_type((lo & 0xFFFF) | (hi << 16), jnp.float32)


def _unpack32(wf):
    w = lax.bitcast_convert_type(wf, jnp.int32)
    lo = lax.bitcast_convert_type(w.astype(jnp.int16), jnp.bfloat16)
    hi = lax.bitcast_convert_type((w >> 16).astype(jnp.int16), jnp.bfloat16)
    return lo.astype(jnp.float32), hi.astype(jnp.float32)


def _precompute_body(me_ref, ge_ref, w1a_ref, w1b_ref, b1_ref, a_ref, b_ref):
    a = jnp.dot(me_ref[...].astype(jnp.bfloat16), w1a_ref[...],
                preferred_element_type=jnp.float32).astype(jnp.bfloat16)
    a_ref[...] = _pack32(a)
    b = (jnp.dot(ge_ref[...].astype(jnp.bfloat16), w1b_ref[...],
                 preferred_element_type=jnp.float32)
         + b1_ref[...]).astype(jnp.bfloat16)
    b_ref[...] = _pack32(b)


def _precompute(mesh_e, grid_embed, w1a, w1b, b1, block_n=2048):
    n = mesh_e.shape[0]
    d = _DIM
    full = lambda shape: pl.BlockSpec(shape, lambda i: (0, 0))
    return pl.pallas_call(
        _precompute_body,
        grid=(n // block_n,),
        in_specs=[
            pl.BlockSpec((block_n, d), lambda i: (i, 0)),
            pl.BlockSpec((block_n, d), lambda i: (i, 0)),
            full((d, 2 * d)),
            full((d, 2 * d)),
            full((1, 2 * d)),
        ],
        out_specs=[pl.BlockSpec((block_n, d), lambda i: (i, 0)),
                   pl.BlockSpec((block_n, d), lambda i: (i, 0))],
        out_shape=[jax.ShapeDtypeStruct((n, d), jnp.float32),
                   jax.ShapeDtypeStruct((n, d), jnp.float32)],
    )(mesh_e, grid_embed, w1a.astype(jnp.bfloat16), w1b.astype(jnp.bfloat16),
      b1.reshape(1, -1))
